# baseline (speedup 1.0000x reference)
import functools

import jax
import jax.numpy as jnp
from jax import lax
from jax.experimental import pallas as pl
from jax.experimental.pallas import tpu as pltpu
from jax.experimental.pallas import tpu_sc as plsc

BATCH = 16384
EMBED = 32
HIDDEN = 64
ROLES = 16
PEDS = 8

NC = 2
NS = 16
NW = NC * NS
BPW = BATCH // NW
CH = 64
NCHUNK = BPW // CH
LANES = 16
WIDE = 128
PACK = WIDE // EMBED


def _fill_wide_idx(idx_v, widx_v):
    def body(g, carry):
        o = g * LANES
        widx_v[pl.ds(o, LANES)] = lax.shift_right_logical(
            idx_v[pl.ds(o, LANES)], 2)
        return carry
    lax.fori_loop(0, BPW // LANES, body, 0)


def _extract_chunk(idx_v, wide_v, out_v, c):
    iota = lax.iota(jnp.int32, LANES)

    def body(g, carry):
        rows = g * LANES + iota
        grows = c * CH + rows
        off = (idx_v[pl.ds(c * CH + g * LANES, LANES)] & 3) * EMBED
        orow = lax.shift_right_logical(grows, 2)
        ocol0 = (grows & 3) * EMBED
        for w in range(EMBED):
            val = plsc.load_gather(wide_v, [rows, off + w])
            plsc.store_scatter(out_v, [orow, ocol0 + w], val)
        return carry
    lax.fori_loop(0, CH // LANES, body, 0)


def _gather_one_table(base, idx_hbm, tabw_hbm, embw_hbm,
                      idx_v, widx_v, wide0_v, wide1_v, out_v, sem0, sem1):
    pltpu.sync_copy(idx_hbm.at[pl.ds(base, BPW)], idx_v)
    _fill_wide_idx(idx_v, widx_v)
    bufs = (wide0_v, wide1_v)
    sems = (sem0, sem1)
    dmas = [None, None]
    for c in range(2):
        dmas[c] = pltpu.async_copy(
            tabw_hbm.at[widx_v.at[pl.ds(c * CH, CH)]], bufs[c], sems[c])
    for c in range(NCHUNK):
        dmas[c % 2].wait()
        _extract_chunk(idx_v, bufs[c % 2], out_v, c)
        if c + 2 < NCHUNK:
            nxt = c + 2
            dmas[nxt % 2] = pltpu.async_copy(
                tabw_hbm.at[widx_v.at[pl.ds(nxt * CH, CH)]],
                bufs[nxt % 2], sems[nxt % 2])
    pltpu.sync_copy(
        out_v,
        embw_hbm.at[pl.ds(pl.multiple_of(base // PACK, BPW // PACK),
                          BPW // PACK)])


def _gather_body(cidx_hbm, sidx_hbm, ctabw_hbm, stabw_hbm,
                 cembw_hbm, sembw_hbm,
                 cidx_v, sidx_v, widx_v,
                 cwide0_v, cwide1_v, swide0_v, swide1_v,
                 cout_v, sout_v, sem0, sem1, sem2, sem3):
    wid = lax.axis_index("s") * NC + lax.axis_index("c")
    base = pl.multiple_of(wid * BPW, BPW)
    _gather_one_table(base, cidx_hbm, ctabw_hbm, cembw_hbm,
                      cidx_v, widx_v, cwide0_v, cwide1_v, cout_v, sem0, sem1)
    _gather_one_table(base, sidx_hbm, stabw_hbm, sembw_hbm,
                      sidx_v, widx_v, swide0_v, swide1_v, sout_v, sem2, sem3)


@functools.cache
def _make_gather():
    return pl.kernel(
        _gather_body,
        out_type=(
            jax.ShapeDtypeStruct((BATCH // PACK, WIDE), jnp.float32),
            jax.ShapeDtypeStruct((BATCH // PACK, WIDE), jnp.float32),
        ),
        mesh=plsc.VectorSubcoreMesh(
            core_axis_name="c", subcore_axis_name="s",
            num_cores=NC, num_subcores=NS,
        ),
        scratch_types=[
            pltpu.VMEM((BPW,), jnp.int32),
            pltpu.VMEM((BPW,), jnp.int32),
            pltpu.VMEM((BPW,), jnp.int32),
            pltpu.VMEM((CH, WIDE), jnp.float32),
            pltpu.VMEM((CH, WIDE), jnp.float32),
            pltpu.VMEM((CH, WIDE), jnp.float32),
            pltpu.VMEM((CH, WIDE), jnp.float32),
            pltpu.VMEM((BPW // PACK, WIDE), jnp.float32),
            pltpu.VMEM((BPW // PACK, WIDE), jnp.float32),
            pltpu.SemaphoreType.DMA,
            pltpu.SemaphoreType.DMA,
            pltpu.SemaphoreType.DMA,
            pltpu.SemaphoreType.DMA,
        ],
        compiler_params=pltpu.CompilerParams(needs_layout_passes=False),
    )


BLK4 = 512


def _mlp_body(c_ref, s_ref, wh_ref, bh_ref, wr_ref, br_ref, wp_ref, bp_ref,
              role_ref, ped_ref):
    cw = c_ref[...]
    sw = s_ref[...]
    wh = wh_ref[...]
    top = wh[:EMBED, :]
    bot = wh[EMBED:, :]
    for k in range(PACK):
        c = cw[:, k * EMBED:(k + 1) * EMBED]
        s = sw[:, k * EMBED:(k + 1) * EMBED]
        h = jnp.dot(c, top, preferred_element_type=jnp.float32)
        h = h + jnp.dot(s, bot, preferred_element_type=jnp.float32)
        h = jnp.maximum(h + bh_ref[...], 0.0)
        role_ref[:, k * ROLES:(k + 1) * ROLES] = (
            jnp.dot(h, wr_ref[...], preferred_element_type=jnp.float32)
            + br_ref[...])
        ped_ref[:, k * PEDS:(k + 1) * PEDS] = (
            jnp.dot(h, wp_ref[...], preferred_element_type=jnp.float32)
            + bp_ref[...])


def _mlp(cembw, sembw, W_h, b_h2, W_r, b_r2, W_p, b_p2, interpret=False):
    rep = lambda shape: pl.BlockSpec(shape, lambda i: (0, 0))
    nwide = BATCH // PACK
    return pl.pallas_call(
        _mlp_body,
        grid=(nwide // BLK4,),
        in_specs=[
            pl.BlockSpec((BLK4, WIDE), lambda i: (i, 0)),
            pl.BlockSpec((BLK4, WIDE), lambda i: (i, 0)),
            rep((2 * EMBED, HIDDEN)),
            rep((1, HIDDEN)),
            rep((HIDDEN, ROLES)),
            rep((1, ROLES)),
            rep((HIDDEN, PEDS)),
            rep((1, PEDS)),
        ],
        out_specs=[
            pl.BlockSpec((BLK4, PACK * ROLES), lambda i: (i, 0)),
            pl.BlockSpec((BLK4, PACK * PEDS), lambda i: (i, 0)),
        ],
        out_shape=[
            jax.ShapeDtypeStruct((nwide, PACK * ROLES), jnp.float32),
            jax.ShapeDtypeStruct((nwide, PACK * PEDS), jnp.float32),
        ],
        interpret=interpret,
    )(cembw, sembw, W_h, b_h2, W_r, b_r2, W_p, b_p2)


def kernel(concept_idx, style_idx, concept_table, style_table,
           W_h, b_h, W_r, b_r, W_p, b_p):
    ctabw = concept_table.reshape(-1, WIDE)
    stabw = style_table.reshape(-1, WIDE)
    cembw, sembw = _make_gather()(concept_idx.astype(jnp.int32),
                                  style_idx.astype(jnp.int32),
                                  ctabw, stabw)
    role_w, ped_w = _mlp(cembw, sembw, W_h, b_h.reshape(1, HIDDEN),
                         W_r, b_r.reshape(1, ROLES),
                         W_p, b_p.reshape(1, PEDS))
    return (role_w.reshape(BATCH, ROLES), ped_w.reshape(BATCH, PEDS))


# speedup vs baseline: 1.7383x; 1.7383x over previous
import functools

import jax
import jax.numpy as jnp
from jax import lax
from jax.experimental import pallas as pl
from jax.experimental.pallas import tpu as pltpu
from jax.experimental.pallas import tpu_sc as plsc

BATCH = 16384
EMBED = 32
HIDDEN = 64
ROLES = 16
PEDS = 8

NC = 2
NS = 16
NW = NC * NS
BPW = BATCH // NW
CH = 32
NCHUNK = BPW // CH
LANES = 16
WIDE = 128
PACK = WIDE // EMBED


TILE_H = 8


def _process_group(idx_v, tab3_hbm, buf_v, out_v, sem, g):
    iota = lax.iota(jnp.int32, LANES)
    vec = idx_v[pl.ds(g * LANES, LANES)]
    tvec = lax.shift_right_logical(vec, 3)
    for j in range(LANES):
        tj = tvec[j]
        pltpu.async_copy(tab3_hbm.at[tj],
                         buf_v.at[pl.ds(j * TILE_H, TILE_H)], sem)
    for j in range(LANES):
        pltpu.make_async_copy(
            tab3_hbm.at[0], buf_v.at[pl.ds(j * TILE_H, TILE_H)], sem).wait()
    grows = g * LANES + iota
    sub = vec & 7
    rowv = iota * TILE_H + sub
    orow = lax.shift_right_logical(grows, 2)
    ocol0 = (grows & 3) * EMBED
    for w in range(EMBED):
        wv = jnp.full((LANES,), w, jnp.int32)
        val = plsc.load_gather(buf_v, [rowv, wv])
        plsc.store_scatter(out_v, [orow, ocol0 + w], val)


def _gather_body(cidx_hbm, sidx_hbm, ctab3_hbm, stab3_hbm,
                 cembw_hbm, sembw_hbm,
                 cidx_v, sidx_v,
                 cbuf_v, sbuf_v,
                 cout_v, sout_v, sem_c, sem_s):
    wid = lax.axis_index("s") * NC + lax.axis_index("c")
    base = pl.multiple_of(wid * BPW, BPW)
    pltpu.sync_copy(cidx_hbm.at[pl.ds(base, BPW)], cidx_v)
    pltpu.sync_copy(sidx_hbm.at[pl.ds(base, BPW)], sidx_v)

    def body(g, carry):
        _process_group(cidx_v, ctab3_hbm, cbuf_v, cout_v, sem_c, g)
        _process_group(sidx_v, stab3_hbm, sbuf_v, sout_v, sem_s, g)
        return carry
    lax.fori_loop(0, BPW // LANES, body, 0)

    obase = pl.multiple_of(base // PACK, BPW // PACK)
    pltpu.sync_copy(cout_v, cembw_hbm.at[pl.ds(obase, BPW // PACK)])
    pltpu.sync_copy(sout_v, sembw_hbm.at[pl.ds(obase, BPW // PACK)])


@functools.cache
def _make_gather():
    return pl.kernel(
        _gather_body,
        out_type=(
            jax.ShapeDtypeStruct((BATCH // PACK, WIDE), jnp.float32),
            jax.ShapeDtypeStruct((BATCH // PACK, WIDE), jnp.float32),
        ),
        mesh=plsc.VectorSubcoreMesh(
            core_axis_name="c", subcore_axis_name="s",
            num_cores=NC, num_subcores=NS,
        ),
        scratch_types=[
            pltpu.VMEM((BPW,), jnp.int32),
            pltpu.VMEM((BPW,), jnp.int32),
            pltpu.VMEM((LANES * TILE_H, EMBED), jnp.float32),
            pltpu.VMEM((LANES * TILE_H, EMBED), jnp.float32),
            pltpu.VMEM((BPW // PACK, WIDE), jnp.float32),
            pltpu.VMEM((BPW // PACK, WIDE), jnp.float32),
            pltpu.SemaphoreType.DMA,
            pltpu.SemaphoreType.DMA,
        ],
        compiler_params=pltpu.CompilerParams(needs_layout_passes=False),
    )


BLK4 = 512


def _mlp_body(c_ref, s_ref, wh_ref, bh_ref, wr_ref, br_ref, wp_ref, bp_ref,
              role_ref, ped_ref):
    cw = c_ref[...]
    sw = s_ref[...]
    wh = wh_ref[...]
    top = wh[:EMBED, :]
    bot = wh[EMBED:, :]
    for k in range(PACK):
        c = cw[:, k * EMBED:(k + 1) * EMBED]
        s = sw[:, k * EMBED:(k + 1) * EMBED]
        h = jnp.dot(c, top, preferred_element_type=jnp.float32)
        h = h + jnp.dot(s, bot, preferred_element_type=jnp.float32)
        h = jnp.maximum(h + bh_ref[...], 0.0)
        role_ref[:, k * ROLES:(k + 1) * ROLES] = (
            jnp.dot(h, wr_ref[...], preferred_element_type=jnp.float32)
            + br_ref[...])
        ped_ref[:, k * PEDS:(k + 1) * PEDS] = (
            jnp.dot(h, wp_ref[...], preferred_element_type=jnp.float32)
            + bp_ref[...])


def _mlp(cembw, sembw, W_h, b_h2, W_r, b_r2, W_p, b_p2, interpret=False):
    rep = lambda shape: pl.BlockSpec(shape, lambda i: (0, 0))
    nwide = BATCH // PACK
    return pl.pallas_call(
        _mlp_body,
        grid=(nwide // BLK4,),
        in_specs=[
            pl.BlockSpec((BLK4, WIDE), lambda i: (i, 0)),
            pl.BlockSpec((BLK4, WIDE), lambda i: (i, 0)),
            rep((2 * EMBED, HIDDEN)),
            rep((1, HIDDEN)),
            rep((HIDDEN, ROLES)),
            rep((1, ROLES)),
            rep((HIDDEN, PEDS)),
            rep((1, PEDS)),
        ],
        out_specs=[
            pl.BlockSpec((BLK4, PACK * ROLES), lambda i: (i, 0)),
            pl.BlockSpec((BLK4, PACK * PEDS), lambda i: (i, 0)),
        ],
        out_shape=[
            jax.ShapeDtypeStruct((nwide, PACK * ROLES), jnp.float32),
            jax.ShapeDtypeStruct((nwide, PACK * PEDS), jnp.float32),
        ],
        interpret=interpret,
    )(cembw, sembw, W_h, b_h2, W_r, b_r2, W_p, b_p2)


def kernel(concept_idx, style_idx, concept_table, style_table,
           W_h, b_h, W_r, b_r, W_p, b_p):
    ctab3 = concept_table.reshape(-1, TILE_H, EMBED)
    stab3 = style_table.reshape(-1, TILE_H, EMBED)
    cembw, sembw = _make_gather()(concept_idx.astype(jnp.int32),
                                  style_idx.astype(jnp.int32),
                                  ctab3, stab3)
    role_w, ped_w = _mlp(cembw, sembw, W_h, b_h.reshape(1, HIDDEN),
                         W_r, b_r.reshape(1, ROLES),
                         W_p, b_p.reshape(1, PEDS))
    return (role_w.reshape(BATCH, ROLES), ped_w.reshape(BATCH, PEDS))
